# baseline (speedup 1.0000x reference)
import functools

import jax
import jax.numpy as jnp
from jax import lax
from jax.experimental import pallas as pl
from jax.experimental.pallas import tpu as pltpu
from jax.experimental.pallas import tpu_sc as plsc

B, L, D = 4096, 200, 64
TOT = B * L
NW = 32
BPW = TOT // NW
HPW = BPW // 2
C = 512
CH = C // 2
NCHUNK = BPW // C

_MESH = plsc.VectorSubcoreMesh(core_axis_name="c", subcore_axis_name="s")


@functools.partial(
    pl.kernel,
    mesh=_MESH,
    out_type=jax.ShapeDtypeStruct((TOT // 2, 2 * D), jnp.float32),
    scratch_types=[
        pltpu.VMEM((HPW,), jnp.int32),
        pltpu.VMEM((HPW,), jnp.int32),
        pltpu.VMEM((CH, D), jnp.float32),
        pltpu.VMEM((CH, D), jnp.float32),
        pltpu.VMEM((CH, D), jnp.float32),
        pltpu.VMEM((CH, D), jnp.float32),
        pltpu.SemaphoreType.DMA,
        pltpu.SemaphoreType.DMA,
        pltpu.SemaphoreType.DMA,
        pltpu.SemaphoreType.DMA,
    ],
    compiler_params=pltpu.CompilerParams(use_tc_tiling_on_sc=False),
)
def _emb_gather(idx_e_hbm, idx_o_hbm, table_hbm, out_hbm,
                slab_e, slab_o, re0, ro0, re1, ro1, g0, g1, o0, o1):
    wid = lax.axis_index("s") * 2 + lax.axis_index("c")
    hbase = wid * HPW
    pltpu.sync_copy(idx_e_hbm.at[pl.ds(hbase, HPW)], slab_e)
    pltpu.sync_copy(idx_o_hbm.at[pl.ds(hbase, HPW)], slab_o)

    def gather(ci, re, ro, g):
        pltpu.async_copy(table_hbm.at[slab_e.at[pl.ds(ci * CH, CH)]], re, g)
        pltpu.async_copy(table_hbm.at[slab_o.at[pl.ds(ci * CH, CH)]], ro, g)

    def store(ci, re, ro, o):
        r0 = hbase + ci * CH
        pltpu.async_copy(re, out_hbm.at[pl.ds(r0, CH), pl.ds(0, D)], o)
        pltpu.async_copy(ro, out_hbm.at[pl.ds(r0, CH), pl.ds(D, D)], o)

    def wait_gather(re, ro, g):
        pltpu.make_async_copy(table_hbm.at[slab_e.at[pl.ds(0, CH)]], re, g).wait()
        pltpu.make_async_copy(table_hbm.at[slab_o.at[pl.ds(0, CH)]], ro, g).wait()

    def wait_store(re, ro, o):
        pltpu.make_async_copy(re, out_hbm.at[pl.ds(hbase, CH), pl.ds(0, D)], o).wait()
        pltpu.make_async_copy(ro, out_hbm.at[pl.ds(hbase, CH), pl.ds(D, D)], o).wait()

    gather(0, re0, ro0, g0)
    gather(1, re1, ro1, g1)
    wait_gather(re0, ro0, g0)
    store(0, re0, ro0, o0)
    wait_gather(re1, ro1, g1)
    store(1, re1, ro1, o1)

    def body(k, _):
        c0 = 2 * k
        wait_store(re0, ro0, o0)
        gather(c0, re0, ro0, g0)
        wait_store(re1, ro1, o1)
        gather(c0 + 1, re1, ro1, g1)
        wait_gather(re0, ro0, g0)
        store(c0, re0, ro0, o0)
        wait_gather(re1, ro1, g1)
        store(c0 + 1, re1, ro1, o1)
        return 0

    lax.fori_loop(1, NCHUNK // 2, body, 0)
    wait_store(re0, ro0, o0)
    wait_store(re1, ro1, o1)


def kernel(x, table):
    flat = x.reshape(TOT).astype(jnp.int32)
    idx_e = flat[0::2]
    idx_o = flat[1::2]
    out = _emb_gather(idx_e, idx_o, table)
    return out.reshape(B, L, D)
